# baseline (speedup 1.0000x reference)
import functools

import jax
import jax.numpy as jnp
from jax import lax
from jax.experimental import pallas as pl
from jax.experimental.pallas import tpu as pltpu, tpu_sc as plsc


def _make_gather(V, D, B):
    info = plsc.get_sparse_core_info()
    NC, NS = info.num_cores, info.num_subcores
    NW = NC * NS
    assert B % (8 * NW) == 0
    b_per_w = B // NW
    mesh = plsc.VectorSubcoreMesh(core_axis_name="c", subcore_axis_name="s")

    @functools.partial(
        pl.kernel,
        mesh=mesh,
        out_type=jax.ShapeDtypeStruct((B, D), jnp.float32),
        scratch_types=[
            pltpu.VMEM((b_per_w,), jnp.int32),
            pltpu.VMEM((b_per_w, D), jnp.float32),
            pltpu.SemaphoreType.DMA,
        ],
    )
    def k(table_hbm, idx_hbm, out_hbm, idx_v, rows_v, sem):
        wid = lax.axis_index("s") * NC + lax.axis_index("c")
        base = wid * b_per_w
        pltpu.sync_copy(idx_hbm.at[pl.ds(base, b_per_w)], idx_v)

        U = 16

        def fire(c, carry):
            iv = idx_v[pl.ds(c * U, U)]
            for u in range(U):
                r = c * U + u
                pltpu.async_copy(
                    table_hbm.at[pl.ds(iv[u], 1)], rows_v.at[pl.ds(r, 1)], sem
                )
            return carry

        lax.fori_loop(0, b_per_w // U, fire, 0)
        pltpu.make_async_copy(table_hbm.at[pl.ds(0, b_per_w)], rows_v, sem).wait()
        pltpu.sync_copy(rows_v, out_hbm.at[pl.ds(base, b_per_w)])

    return k


def kernel(labels, embed_table):
    B = labels.shape[0]
    V, D = embed_table.shape
    gather = _make_gather(V, D, B)
    return gather(embed_table, labels.astype(jnp.int32))
